# baseline (speedup 1.0000x reference)
import jax
import jax.numpy as jnp
from jax.experimental import pallas as pl
from jax.experimental.pallas import tpu as pltpu

_TB = 256


def _banded_conv_weights(w, w_in):
    cout, cin, kh, kw = w.shape
    half = (w_in - kw + 1) // 2
    win = jnp.arange(w_in)[:, None]
    halves = []
    for parity in (0, 1):
        kwi = win - (2 * jnp.arange(half)[None, :] + parity)
        valid = (kwi >= 0) & (kwi < kw)
        g = w[:, :, :, jnp.clip(kwi, 0, kw - 1)]
        g = jnp.where(valid[None, None, None], g, 0.0)
        g = jnp.transpose(g, (2, 3, 1, 4, 0))
        halves.append(g.reshape(kh, w_in * cin, half * cout))
    return jnp.concatenate(halves, axis=2)


def _conv1_weights(conv1_w):
    band = _banded_conv_weights(conv1_w, 32)
    z = jnp.zeros((32, 84), jnp.float32)

    def qblock(half, c, r, s):
        blocks = []
        for q in range(4):
            ki = 4 * s + q - r
            blocks.append(band[ki, c::3, 84 * half:84 * (half + 1)]
                          if 0 <= ki <= 4 else z)
        return jnp.concatenate(blocks, axis=0)

    ws = []
    for rho in (0, 1):
        for s in (0, 1):
            for c in range(3):
                ws.append(jnp.concatenate(
                    [qblock(0, c, rho, s), qblock(0, c, rho + 2, s),
                     qblock(1, c, rho, s), qblock(1, c, rho + 2, s)],
                    axis=1))
    w1s = jnp.stack(ws).astype(jnp.bfloat16)
    w1 = jnp.concatenate(
        [w1s[:6].reshape(768, 336), w1s[6:].reshape(768, 336)], axis=1)
    return jnp.pad(w1.reshape(768, 8, 84),
                   ((0, 0), (0, 0), (0, 44))).reshape(768, 1024)


def _conv2_weights(conv2_w):
    band = _banded_conv_weights(conv2_w, 14)
    z = jnp.zeros((84, 160), jnp.float32)

    def tap(ki):
        return band[ki] if 0 <= ki <= 4 else z

    ws = []
    for rho in (0, 1):
        for s in range(3):
            k0 = 2 * s - rho
            ws.append(jnp.concatenate([tap(k0), tap(k0 + 1)], axis=0))
    w2s = jnp.stack(ws).astype(jnp.bfloat16)
    w2 = jnp.concatenate(
        [w2s[:3].reshape(504, 160), w2s[3:].reshape(504, 160)], axis=1)
    w2 = jnp.pad(w2.reshape(504, 4, 80),
                 ((0, 0), (0, 0), (0, 48))).reshape(6, 84, 512)
    return jnp.pad(w2, ((0, 0), (0, 44), (0, 0))).reshape(768, 512)


def _lenet_body(x0_ref, x1_ref, x2_ref, w1_ref, b1_ref, w2_ref, b2_ref,
                wf1_ref, bf1_ref, wf2_ref, bf2_ref, wf3_ref, bf3_ref,
                out_ref):
    f32 = jnp.float32
    bf16 = jnp.bfloat16
    xc = [r[...].reshape(_TB * 8, 128).astype(bf16)
          for r in (x0_ref, x1_ref, x2_ref)]

    l1 = _TB * 8 - 1
    xcat = jnp.concatenate(
        [xc[0][0:l1], xc[1][0:l1], xc[2][0:l1],
         xc[0][1:1 + l1], xc[1][1:1 + l1], xc[2][1:1 + l1]], axis=1)
    uv = jnp.dot(xcat, w1_ref[...], preferred_element_type=f32)
    w = jnp.maximum(uv[:, :512], uv[:, 512:])
    pre = jnp.maximum(w[:, :256], w[:, 256:])
    p1 = jnp.maximum(pre + b1_ref[...], 0.0).astype(bf16)

    l2 = l1 - 2
    pcat = jnp.concatenate([p1[0:l2], p1[1:1 + l2], p1[2:2 + l2]], axis=1)
    uv = jnp.dot(pcat, w2_ref[...], preferred_element_type=f32)
    pre = jnp.maximum(uv[:, :256], uv[:, 256:])
    pre = jnp.maximum(pre[:, :128], pre[:, 128:])
    p2 = jnp.maximum(pre + b2_ref[...], 0.0).astype(bf16)

    l3 = l2 - 4
    fcat = jnp.concatenate([p2[h:h + l3] for h in range(5)], axis=1)
    acc = jnp.dot(fcat, wf1_ref[...], preferred_element_type=f32)
    f1 = jnp.maximum(acc + bf1_ref[...], 0.0).astype(bf16)

    f2 = jnp.maximum(jnp.dot(f1, wf2_ref[...], preferred_element_type=f32)
                     + bf2_ref[...], 0.0).astype(bf16)
    f3 = (jnp.dot(f2, wf3_ref[...], preferred_element_type=f32)
          + bf3_ref[...])
    rows = jax.lax.broadcasted_iota(jnp.int32, (_TB, l3), 0)
    cols = jax.lax.broadcasted_iota(jnp.int32, (_TB, l3), 1)
    sel = (cols == 8 * rows).astype(f32)
    out_ref[...] = jnp.dot(sel, f3, preferred_element_type=f32)


@jax.jit
def _forward(conv1_w, conv1_b, conv2_w, conv2_b, fc1_w, fc1_b,
             fc2_w, fc2_b, fc3_w, fc3_b, x):
    n = x.shape[0]
    n_pad = -(-n // _TB) * _TB
    xr = x.astype(jnp.float32)
    if n_pad != n:
        xr = jnp.pad(xr, ((0, n_pad - n), (0, 0), (0, 0), (0, 0)))
    xq = xr.reshape(n_pad, 3, 8, 128)

    w1 = _conv1_weights(conv1_w)
    b1 = jnp.pad(jnp.tile(conv1_b, 28).reshape(2, 84),
                 ((0, 0), (0, 44))).reshape(1, 256)
    w2 = _conv2_weights(conv2_w)
    b2 = jnp.pad(jnp.tile(conv2_b, 5)[None, :], ((0, 0), (0, 48)))
    wf1 = fc1_w.reshape(120, 16, 5, 5).transpose(2, 3, 1, 0).reshape(
        5, 80, 120).astype(jnp.bfloat16)
    wf1 = jnp.pad(wf1, ((0, 0), (0, 48), (0, 0))).reshape(640, 120)
    bf1 = fc1_b[None, :]
    wf2 = fc2_w.T.astype(jnp.bfloat16)
    bf2 = fc2_b[None, :]
    wf3 = fc3_w.T.astype(jnp.bfloat16)
    bf3 = fc3_b[None, :]

    def w2d(shape):
        return pl.BlockSpec(shape, lambda b: (0, 0))

    out = pl.pallas_call(
        _lenet_body,
        out_shape=jax.ShapeDtypeStruct((n_pad, 10), jnp.float32),
        grid=(n_pad // _TB,),
        in_specs=[
            pl.BlockSpec((_TB, 1, 8, 128), lambda b: (b, 0, 0, 0)),
            pl.BlockSpec((_TB, 1, 8, 128), lambda b: (b, 1, 0, 0)),
            pl.BlockSpec((_TB, 1, 8, 128), lambda b: (b, 2, 0, 0)),
            w2d((768, 1024)), w2d((1, 256)),
            w2d((768, 512)), w2d((1, 128)),
            w2d((640, 120)), w2d((1, 120)),
            w2d((120, 84)), w2d((1, 84)),
            w2d((84, 10)), w2d((1, 10)),
        ],
        out_specs=pl.BlockSpec((_TB, 10), lambda b: (b, 0)),
        compiler_params=pltpu.CompilerParams(
            dimension_semantics=("parallel",)),
    )(xq, xq, xq, w1, b1, w2, b2, wf1, bf1, wf2, bf2, wf3, bf3)
    return out[:n]


def kernel(conv1_w, conv1_b, conv2_w, conv2_b, fc1_w, fc1_b,
           fc2_w, fc2_b, fc3_w, fc3_b, x):
    return _forward(conv1_w, conv1_b, conv2_w, conv2_b, fc1_w, fc1_b,
                    fc2_w, fc2_b, fc3_w, fc3_b, x)


# speedup vs baseline: 1.0237x; 1.0237x over previous
import jax
import jax.numpy as jnp
from jax.experimental import pallas as pl
from jax.experimental.pallas import tpu as pltpu

_TB = 256


def _banded_conv_weights(w, w_in):
    cout, cin, kh, kw = w.shape
    half = (w_in - kw + 1) // 2
    win = jnp.arange(w_in)[:, None]
    halves = []
    for parity in (0, 1):
        kwi = win - (2 * jnp.arange(half)[None, :] + parity)
        valid = (kwi >= 0) & (kwi < kw)
        g = w[:, :, :, jnp.clip(kwi, 0, kw - 1)]
        g = jnp.where(valid[None, None, None], g, 0.0)
        g = jnp.transpose(g, (2, 3, 1, 4, 0))
        halves.append(g.reshape(kh, w_in * cin, half * cout))
    return jnp.concatenate(halves, axis=2)


def _conv1_weights(conv1_w):
    band = _banded_conv_weights(conv1_w, 32)
    z = jnp.zeros((32, 84), jnp.float32)

    def qblock(half, c, r, s):
        blocks = []
        for q in range(4):
            ki = 4 * s + q - r
            blocks.append(band[ki, c::3, 84 * half:84 * (half + 1)]
                          if 0 <= ki <= 4 else z)
        return jnp.concatenate(blocks, axis=0)

    ws = []
    for rho in (0, 1):
        for s in (0, 1):
            for c in range(3):
                ws.append(jnp.concatenate(
                    [qblock(0, c, rho, s), qblock(0, c, rho + 2, s),
                     qblock(1, c, rho, s), qblock(1, c, rho + 2, s)],
                    axis=1))
    w1s = jnp.stack(ws).astype(jnp.bfloat16)
    w1 = jnp.concatenate(
        [w1s[:6].reshape(768, 336), w1s[6:].reshape(768, 336)], axis=1)
    return jnp.pad(w1.reshape(768, 8, 84),
                   ((0, 0), (0, 0), (0, 44))).reshape(768, 1024)


def _conv2_weights(conv2_w):
    band = _banded_conv_weights(conv2_w, 14)
    z = jnp.zeros((84, 160), jnp.float32)

    def tap(ki):
        return band[ki] if 0 <= ki <= 4 else z

    ws = []
    for rho in (0, 1):
        for s in range(3):
            k0 = 2 * s - rho
            ws.append(jnp.concatenate([tap(k0), tap(k0 + 1)], axis=0))
    w2s = jnp.stack(ws).astype(jnp.bfloat16)
    w2 = jnp.concatenate(
        [w2s[:3].reshape(504, 160), w2s[3:].reshape(504, 160)], axis=1)
    w2 = jnp.pad(w2.reshape(504, 4, 80),
                 ((0, 0), (0, 0), (0, 48))).reshape(6, 84, 512)
    return jnp.pad(w2, ((0, 0), (0, 44), (0, 0))).reshape(768, 512)


def _lenet_body(x0_ref, x1_ref, x2_ref, w1_ref, b1_ref, w2_ref, b2_ref,
                wf1_ref, bf1_ref, wf2_ref, bf2_ref, wf3_ref, bf3_ref,
                out_ref):
    f32 = jnp.float32
    bf16 = jnp.bfloat16
    xc = [r[...].reshape(_TB * 8, 128).astype(bf16)
          for r in (x0_ref, x1_ref, x2_ref)]

    l1 = _TB * 8 - 1
    xcat = jnp.concatenate(
        [xc[0][0:l1], xc[1][0:l1], xc[2][0:l1],
         xc[0][1:1 + l1], xc[1][1:1 + l1], xc[2][1:1 + l1]], axis=1)
    uv = jnp.dot(xcat, w1_ref[...], preferred_element_type=f32)
    w = jnp.maximum(uv[:, :512], uv[:, 512:])
    pre = jnp.maximum(w[:, :256], w[:, 256:])
    p1 = jnp.maximum(pre + b1_ref[...], 0.0).astype(bf16)

    l2 = l1 - 2
    pcat = jnp.concatenate([p1[0:l2], p1[1:1 + l2], p1[2:2 + l2]], axis=1)
    uv = jnp.dot(pcat, w2_ref[...], preferred_element_type=f32)
    pre = jnp.maximum(uv[:, :256], uv[:, 256:])
    pre = jnp.maximum(pre[:, :128], pre[:, 128:])
    p2 = jnp.maximum(pre + b2_ref[...], 0.0).astype(bf16)

    l3 = l2 - 4
    fcat = jnp.concatenate([p2[h:h + l3] for h in range(5)], axis=1)
    acc = jnp.dot(fcat, wf1_ref[...], preferred_element_type=f32)
    f1 = jnp.maximum(acc + bf1_ref[...], 0.0).astype(bf16)

    f2 = jnp.maximum(jnp.dot(f1, wf2_ref[...], preferred_element_type=f32)
                     + bf2_ref[...], 0.0).astype(bf16)
    f3 = (jnp.dot(f2, wf3_ref[...], preferred_element_type=f32)
          + bf3_ref[...])
    out_ref[...] = jnp.concatenate(
        [f3, jnp.zeros((_TB * 8 - l3, 10), f32)], axis=0)


@jax.jit
def _forward(conv1_w, conv1_b, conv2_w, conv2_b, fc1_w, fc1_b,
             fc2_w, fc2_b, fc3_w, fc3_b, x):
    n = x.shape[0]
    n_pad = -(-n // _TB) * _TB
    xr = x.astype(jnp.float32)
    if n_pad != n:
        xr = jnp.pad(xr, ((0, n_pad - n), (0, 0), (0, 0), (0, 0)))
    xq = xr.reshape(n_pad, 3, 8, 128)

    w1 = _conv1_weights(conv1_w)
    b1 = jnp.pad(jnp.tile(conv1_b, 28).reshape(2, 84),
                 ((0, 0), (0, 44))).reshape(1, 256)
    w2 = _conv2_weights(conv2_w)
    b2 = jnp.pad(jnp.tile(conv2_b, 5)[None, :], ((0, 0), (0, 48)))
    wf1 = fc1_w.reshape(120, 16, 5, 5).transpose(2, 3, 1, 0).reshape(
        5, 80, 120).astype(jnp.bfloat16)
    wf1 = jnp.pad(wf1, ((0, 0), (0, 48), (0, 0))).reshape(640, 120)
    bf1 = fc1_b[None, :]
    wf2 = fc2_w.T.astype(jnp.bfloat16)
    bf2 = fc2_b[None, :]
    wf3 = fc3_w.T.astype(jnp.bfloat16)
    bf3 = fc3_b[None, :]

    def w2d(shape):
        return pl.BlockSpec(shape, lambda b: (0, 0))

    out = pl.pallas_call(
        _lenet_body,
        out_shape=jax.ShapeDtypeStruct((n_pad * 8, 10), jnp.float32),
        grid=(n_pad // _TB,),
        in_specs=[
            pl.BlockSpec((_TB, 1, 8, 128), lambda b: (b, 0, 0, 0)),
            pl.BlockSpec((_TB, 1, 8, 128), lambda b: (b, 1, 0, 0)),
            pl.BlockSpec((_TB, 1, 8, 128), lambda b: (b, 2, 0, 0)),
            w2d((768, 1024)), w2d((1, 256)),
            w2d((768, 512)), w2d((1, 128)),
            w2d((640, 120)), w2d((1, 120)),
            w2d((120, 84)), w2d((1, 84)),
            w2d((84, 10)), w2d((1, 10)),
        ],
        out_specs=pl.BlockSpec((_TB * 8, 10), lambda b: (b, 0)),
        compiler_params=pltpu.CompilerParams(
            dimension_semantics=("parallel",)),
    )(xq, xq, xq, w1, b1, w2, b2, wf1, bf1, wf2, bf2, wf3, bf3)
    return out.reshape(n_pad, 8, 10)[:n, 0, :]


def kernel(conv1_w, conv1_b, conv2_w, conv2_b, fc1_w, fc1_b,
           fc2_w, fc2_b, fc3_w, fc3_b, x):
    return _forward(conv1_w, conv1_b, conv2_w, conv2_b, fc1_w, fc1_b,
                    fc2_w, fc2_b, fc3_w, fc3_b, x)
